# baseline (speedup 1.0000x reference)
import functools
import jax
import jax.numpy as jnp
from jax import lax
from jax.experimental import pallas as pl
from jax.experimental.pallas import tpu as pltpu
from jax.experimental.pallas import tpu_sc as plsc

H = 128
RELS = ('nmos', 'pmos', 'R', 'L', 'C')
E_REL = (1024, 1024, 768, 640, 640)
OFF = (0, 1024, 2048, 2816, 3456)
E_TOT = 4096
N = 8192
NC, NS, L = 2, 16, 16
NW = NC * NS
SEG_OWN = N // NW
ACC_ROWS = SEG_OWN + 16
CHUNK = 128
N_CHUNKS = E_TOT // CHUNK
G_TOT = 3 * E_TOT
G_PER_TILE = G_TOT // (NC * NS)


def _gelu(x):
    return 0.5 * x * (1.0 + lax.erf(x * 0.7071067811865476))


SEG_BLK = 512


def _tc1_body(t_ref, dc_ref, dstf_ref, *refs):
    w = refs[:32]
    ef1_ref, tab_ref = refs[32], refs[33]
    for r in range(5):
        w1, b1, w2, b2, w3, b3 = w[6 * r:6 * r + 6]
        x = t_ref[OFF[r]:OFF[r] + E_REL[r], :]
        h = _gelu(x * w1[...] + b1[...])
        h = _gelu(lax.dot_general(h, w2[...], (((1,), (1,)), ((), ())),
                                  preferred_element_type=jnp.float32) + b2[...])
        h = lax.dot_general(h, w3[...], (((1,), (1,)), ((), ())),
                            preferred_element_type=jnp.float32) + b3[...]
        ef1_ref[OFF[r]:OFF[r] + E_REL[r], :] = h
    ef1 = ef1_ref[...]
    g = lax.dot_general(ef1, ef1, (((0,), (0,)), ((), ())),
                        preferred_element_type=jnp.float32)
    att = lax.dot_general(ef1, g, (((1,), (0,)), ((), ())),
                          preferred_element_type=jnp.float32)
    ex = jnp.exp(att - jnp.max(att, axis=0, keepdims=True))
    p = jnp.concatenate([ex, ex * ef1], axis=1)
    win, bin_ = w[30], w[31]
    tab_ref[:N, :] = dc_ref[...] * win[...] + bin_[...]
    dsti = dstf_ref[...]
    base_iota = lax.broadcasted_iota(jnp.int32, (1, SEG_BLK), 1)
    p_hi = p.astype(jnp.bfloat16)
    p_lo = (p - p_hi.astype(jnp.float32)).astype(jnp.bfloat16)

    def seg_block(b, _):
        oh = (dsti == base_iota + b * SEG_BLK).astype(jnp.bfloat16)
        seg = lax.dot_general(oh, p_hi, (((0,), (0,)), ((), ())),
                              preferred_element_type=jnp.float32)
        seg += lax.dot_general(oh, p_lo, (((0,), (0,)), ((), ())),
                               preferred_element_type=jnp.float32)
        tab_ref[pl.ds(N + b * SEG_BLK, SEG_BLK), :] = (
            seg[:, H:] / (seg[:, :H] + 1e-16))
        return 0

    lax.fori_loop(0, N // SEG_BLK, seg_block, 0)


def _tc1(t_all, dc_in, dstf, wlist):
    outs = [
        jax.ShapeDtypeStruct((E_TOT, H), jnp.float32),
        jax.ShapeDtypeStruct((2 * N, H), jnp.float32),
    ]
    return pl.pallas_call(_tc1_body, out_shape=outs)(t_all, dc_in, dstf,
                                                     *wlist)


def _scb_body(tab_hbm, gidx_hbm, out_hbm, idxb, rows, sem):
    wid = lax.axis_index("s") * NC + lax.axis_index("c")
    base = wid * G_PER_TILE
    pltpu.sync_copy(gidx_hbm.at[pl.ds(base, G_PER_TILE)], idxb)
    pltpu.async_copy(tab_hbm.at[idxb], rows, sem).wait()
    pltpu.sync_copy(rows, out_hbm.at[pl.ds(base, G_PER_TILE), :])


def _scb(table, gidx):
    mesh = plsc.VectorSubcoreMesh(core_axis_name="c", subcore_axis_name="s",
                                  num_cores=NC, num_subcores=NS)
    f = pl.kernel(
        _scb_body,
        out_type=jax.ShapeDtypeStruct((G_TOT, H), jnp.float32),
        mesh=mesh,
        scratch_types=[
            pltpu.VMEM((G_PER_TILE,), jnp.int32),
            pltpu.VMEM((G_PER_TILE, H), jnp.float32),
            pltpu.SemaphoreType.DMA,
        ],
    )
    return f(table, gidx)


def _tc2_body(g_ref, ef1_ref, *refs):
    out_ref = refs[-1]
    w = refs[:-1]
    ef1 = ef1_ref[...]
    sw = w[40:48]
    h = _gelu(lax.dot_general(ef1, sw[0][...], (((1,), (1,)), ((), ())),
                              preferred_element_type=jnp.float32) + sw[1][...])
    h = _gelu(lax.dot_general(h, sw[2][...], (((1,), (1,)), ((), ())),
                              preferred_element_type=jnp.float32) + sw[3][...])
    h = _gelu(lax.dot_general(h, sw[4][...], (((1,), (1,)), ((), ())),
                              preferred_element_type=jnp.float32) + sw[5][...])
    selfo = lax.dot_general(h, sw[6][...], (((1,), (1,)), ((), ())),
                            preferred_element_type=jnp.float32) + sw[7][...]

    for r in range(5):
        w1, b1, w2, b2, w3, b3, w4, b4 = w[8 * r:8 * r + 8]
        lo, e = OFF[r], E_REL[r]
        src = g_ref[lo:lo + e, :]
        dst = g_ref[E_TOT + lo:E_TOT + lo + e, :]
        ef1_r = ef1_ref[lo:lo + e, :]
        w1m = w1[...]
        if r < 2:
            a = lax.dot_general(src, w1m[:, 0:H], (((1,), (1,)), ((), ())),
                                preferred_element_type=jnp.float32)
            a += lax.dot_general(dst, w1m[:, H:2 * H], (((1,), (1,)), ((), ())),
                                 preferred_element_type=jnp.float32)
            vg = g_ref[2 * E_TOT + lo:2 * E_TOT + lo + e, :]
            vb = g_ref[2 * E_TOT + 2048 + lo:2 * E_TOT + 2048 + lo + e, :]
            a += lax.dot_general(vg, w1m[:, 2 * H:3 * H], (((1,), (1,)), ((), ())),
                                 preferred_element_type=jnp.float32)
            a += lax.dot_general(vb, w1m[:, 3 * H:4 * H], (((1,), (1,)), ((), ())),
                                 preferred_element_type=jnp.float32)
            a += lax.dot_general(ef1_r, w1m[:, 4 * H:5 * H], (((1,), (1,)), ((), ())),
                                 preferred_element_type=jnp.float32)
        else:
            a = lax.dot_general(src, w1m[:, 0:H], (((1,), (1,)), ((), ())),
                                preferred_element_type=jnp.float32)
            a += lax.dot_general(dst, w1m[:, H:2 * H], (((1,), (1,)), ((), ())),
                                 preferred_element_type=jnp.float32)
            a += lax.dot_general(ef1_r, w1m[:, 2 * H:3 * H], (((1,), (1,)), ((), ())),
                                 preferred_element_type=jnp.float32)
        h = _gelu(a + b1[...])
        h = _gelu(lax.dot_general(h, w2[...], (((1,), (1,)), ((), ())),
                                  preferred_element_type=jnp.float32) + b2[...])
        h = _gelu(lax.dot_general(h, w3[...], (((1,), (1,)), ((), ())),
                                  preferred_element_type=jnp.float32) + b3[...])
        h = lax.dot_general(h, w4[...], (((1,), (1,)), ((), ())),
                            preferred_element_type=jnp.float32) + b4[...]
        ef2 = h + selfo[lo:lo + e, :]
        sw_, sb = w[48], w[49]
        out_ref[lo:lo + e, :] = lax.dot_general(
            ef2, sw_[...], (((1,), (1,)), ((), ())),
            preferred_element_type=jnp.float32) + sb[...]


def _tc2(g, ef1, wlist):
    return pl.pallas_call(
        _tc2_body,
        out_shape=jax.ShapeDtypeStruct((E_TOT, H), jnp.float32),
    )(g, ef1, *wlist)


@jax.jit
def kernel(edge_index_nmos, edge_index_pmos, edge_index_R, edge_index_L,
           edge_index_C, edge_attr_nmos, edge_attr_pmos, edge_attr_R,
           edge_attr_L, edge_attr_C, dc_voltages_input, dc_voltages_output,
           weights):
    ei = (edge_index_nmos, edge_index_pmos, edge_index_R, edge_index_L,
          edge_index_C)
    ea = (edge_attr_nmos, edge_attr_pmos, edge_attr_R, edge_attr_L,
          edge_attr_C)

    t_all = jnp.concatenate([a[:, -1:] for a in ea], axis=0)
    src = jnp.concatenate([e[0] for e in ei]).astype(jnp.int32)
    dst = jnp.concatenate([e[1] for e in ei]).astype(jnp.int32)
    vg = jnp.concatenate(
        [(a[:, 0] * N + a[:, 1]).astype(jnp.int32) for a in ea[:2]])
    vb = jnp.concatenate(
        [(a[:, 2] * N + a[:, 3]).astype(jnp.int32) for a in ea[:2]])
    gidx = jnp.concatenate([src, dst + N, vg, vb])

    w = weights
    tc1_w = []
    for r in RELS:
        for (W, b) in w['ef1_' + r]:
            tc1_w.append(W if W.shape[1] > 1 else W.reshape(1, H))
            tc1_w.append(b.reshape(1, H))
    tc1_w.append(w['nf1_input'][0][0].reshape(1, H))
    tc1_w.append(w['nf1_input'][0][1].reshape(1, H))

    ef1, table = _tc1(t_all, dc_voltages_input[:, None], dst[:, None], tc1_w)
    g = _scb(table, gidx)

    tc2_w = []
    for r in RELS:
        for (W, b) in w['ef2_' + r]:
            tc2_w.append(W)
            tc2_w.append(b.reshape(1, H))
    for (W, b) in w['ef2_self']:
        tc2_w.append(W)
        tc2_w.append(b.reshape(1, H))
    tc2_w.append(w['score'][0][0])
    tc2_w.append(w['score'][0][1].reshape(1, H))

    return _tc2(g, ef1, tc2_w)


# speedup vs baseline: 1.1120x; 1.1120x over previous
import functools
import jax
import jax.numpy as jnp
from jax import lax
from jax.experimental import pallas as pl
from jax.experimental.pallas import tpu as pltpu
from jax.experimental.pallas import tpu_sc as plsc

H = 128
RELS = ('nmos', 'pmos', 'R', 'L', 'C')
E_REL = (1024, 1024, 768, 640, 640)
OFF = (0, 1024, 2048, 2816, 3456)
E_TOT = 4096
N = 8192
NC, NS, L = 2, 16, 16
NW = NC * NS
SEG_OWN = N // NW
ACC_ROWS = SEG_OWN + 16
CHUNK = 128
N_CHUNKS = E_TOT // CHUNK
G_TOT = 3 * E_TOT
G_PER_TILE = G_TOT // (NC * NS)


def _gelu(x):
    return 0.5 * x * (1.0 + lax.erf(x * 0.7071067811865476))


SEG_BLK = 512


def _tc1_body(t_ref, dc_ref, dstf_ref, *refs):
    w = refs[:32]
    ef1_ref, tab_ref = refs[32], refs[33]
    for r in range(5):
        w1, b1, w2, b2, w3, b3 = w[6 * r:6 * r + 6]
        x = t_ref[OFF[r]:OFF[r] + E_REL[r], :]
        h = _gelu(x * w1[...] + b1[...])
        h = _gelu(lax.dot_general(h, w2[...], (((1,), (1,)), ((), ())),
                                  preferred_element_type=jnp.float32) + b2[...])
        h = lax.dot_general(h, w3[...], (((1,), (1,)), ((), ())),
                            preferred_element_type=jnp.float32) + b3[...]
        ef1_ref[OFF[r]:OFF[r] + E_REL[r], :] = h
    ef1 = ef1_ref[...]
    g = lax.dot_general(ef1, ef1, (((0,), (0,)), ((), ())),
                        preferred_element_type=jnp.float32)
    att = lax.dot_general(ef1, g, (((1,), (0,)), ((), ())),
                          preferred_element_type=jnp.float32)
    ex = jnp.exp(att - jnp.max(att, axis=0, keepdims=True))
    p = jnp.concatenate([ex, ex * ef1], axis=1)
    win, bin_ = w[30], w[31]
    tab_ref[:N, :] = dc_ref[...] * win[...] + bin_[...]
    dsti = dstf_ref[...]
    base_iota = lax.broadcasted_iota(jnp.int32, (1, SEG_BLK), 1)
    pb = p.astype(jnp.bfloat16)

    def seg_block(b, _):
        oh = (dsti == base_iota + b * SEG_BLK).astype(jnp.bfloat16)
        seg = lax.dot_general(oh, pb, (((0,), (0,)), ((), ())),
                              preferred_element_type=jnp.float32)
        tab_ref[pl.ds(N + b * SEG_BLK, SEG_BLK), :] = (
            seg[:, H:] / (seg[:, :H] + 1e-16))
        return 0

    lax.fori_loop(0, N // SEG_BLK, seg_block, 0)


def _tc1(t_all, dc_in, dstf, wlist):
    outs = [
        jax.ShapeDtypeStruct((E_TOT, H), jnp.float32),
        jax.ShapeDtypeStruct((2 * N, H), jnp.float32),
    ]
    return pl.pallas_call(_tc1_body, out_shape=outs)(t_all, dc_in, dstf,
                                                     *wlist)


def _scb_body(tab_hbm, gidx_hbm, out_hbm, idxb, rows, sem):
    wid = lax.axis_index("s") * NC + lax.axis_index("c")
    base = wid * G_PER_TILE
    pltpu.sync_copy(gidx_hbm.at[pl.ds(base, G_PER_TILE)], idxb)
    pltpu.async_copy(tab_hbm.at[idxb], rows, sem).wait()
    pltpu.sync_copy(rows, out_hbm.at[pl.ds(base, G_PER_TILE), :])


def _scb(table, gidx):
    mesh = plsc.VectorSubcoreMesh(core_axis_name="c", subcore_axis_name="s",
                                  num_cores=NC, num_subcores=NS)
    f = pl.kernel(
        _scb_body,
        out_type=jax.ShapeDtypeStruct((G_TOT, H), jnp.float32),
        mesh=mesh,
        scratch_types=[
            pltpu.VMEM((G_PER_TILE,), jnp.int32),
            pltpu.VMEM((G_PER_TILE, H), jnp.float32),
            pltpu.SemaphoreType.DMA,
        ],
    )
    return f(table, gidx)


def _tc2_body(g_ref, ef1_ref, *refs):
    out_ref = refs[-1]
    w = refs[:-1]
    ef1 = ef1_ref[...]
    sw = w[40:48]
    h = _gelu(lax.dot_general(ef1, sw[0][...], (((1,), (1,)), ((), ())),
                              preferred_element_type=jnp.float32) + sw[1][...])
    h = _gelu(lax.dot_general(h, sw[2][...], (((1,), (1,)), ((), ())),
                              preferred_element_type=jnp.float32) + sw[3][...])
    h = _gelu(lax.dot_general(h, sw[4][...], (((1,), (1,)), ((), ())),
                              preferred_element_type=jnp.float32) + sw[5][...])
    selfo = lax.dot_general(h, sw[6][...], (((1,), (1,)), ((), ())),
                            preferred_element_type=jnp.float32) + sw[7][...]

    for r in range(5):
        w1, b1, w2, b2, w3, b3, w4, b4 = w[8 * r:8 * r + 8]
        lo, e = OFF[r], E_REL[r]
        src = g_ref[lo:lo + e, :]
        dst = g_ref[E_TOT + lo:E_TOT + lo + e, :]
        ef1_r = ef1_ref[lo:lo + e, :]
        w1m = w1[...]
        if r < 2:
            a = lax.dot_general(src, w1m[:, 0:H], (((1,), (1,)), ((), ())),
                                preferred_element_type=jnp.float32)
            a += lax.dot_general(dst, w1m[:, H:2 * H], (((1,), (1,)), ((), ())),
                                 preferred_element_type=jnp.float32)
            vg = g_ref[2 * E_TOT + lo:2 * E_TOT + lo + e, :]
            vb = g_ref[2 * E_TOT + 2048 + lo:2 * E_TOT + 2048 + lo + e, :]
            a += lax.dot_general(vg, w1m[:, 2 * H:3 * H], (((1,), (1,)), ((), ())),
                                 preferred_element_type=jnp.float32)
            a += lax.dot_general(vb, w1m[:, 3 * H:4 * H], (((1,), (1,)), ((), ())),
                                 preferred_element_type=jnp.float32)
            a += lax.dot_general(ef1_r, w1m[:, 4 * H:5 * H], (((1,), (1,)), ((), ())),
                                 preferred_element_type=jnp.float32)
        else:
            a = lax.dot_general(src, w1m[:, 0:H], (((1,), (1,)), ((), ())),
                                preferred_element_type=jnp.float32)
            a += lax.dot_general(dst, w1m[:, H:2 * H], (((1,), (1,)), ((), ())),
                                 preferred_element_type=jnp.float32)
            a += lax.dot_general(ef1_r, w1m[:, 2 * H:3 * H], (((1,), (1,)), ((), ())),
                                 preferred_element_type=jnp.float32)
        h = _gelu(a + b1[...])
        h = _gelu(lax.dot_general(h, w2[...], (((1,), (1,)), ((), ())),
                                  preferred_element_type=jnp.float32) + b2[...])
        h = _gelu(lax.dot_general(h, w3[...], (((1,), (1,)), ((), ())),
                                  preferred_element_type=jnp.float32) + b3[...])
        h = lax.dot_general(h, w4[...], (((1,), (1,)), ((), ())),
                            preferred_element_type=jnp.float32) + b4[...]
        ef2 = h + selfo[lo:lo + e, :]
        sw_, sb = w[48], w[49]
        out_ref[lo:lo + e, :] = lax.dot_general(
            ef2, sw_[...], (((1,), (1,)), ((), ())),
            preferred_element_type=jnp.float32) + sb[...]


def _tc2(g, ef1, wlist):
    return pl.pallas_call(
        _tc2_body,
        out_shape=jax.ShapeDtypeStruct((E_TOT, H), jnp.float32),
    )(g, ef1, *wlist)


@jax.jit
def kernel(edge_index_nmos, edge_index_pmos, edge_index_R, edge_index_L,
           edge_index_C, edge_attr_nmos, edge_attr_pmos, edge_attr_R,
           edge_attr_L, edge_attr_C, dc_voltages_input, dc_voltages_output,
           weights):
    ei = (edge_index_nmos, edge_index_pmos, edge_index_R, edge_index_L,
          edge_index_C)
    ea = (edge_attr_nmos, edge_attr_pmos, edge_attr_R, edge_attr_L,
          edge_attr_C)

    t_all = jnp.concatenate([a[:, -1:] for a in ea], axis=0)
    src = jnp.concatenate([e[0] for e in ei]).astype(jnp.int32)
    dst = jnp.concatenate([e[1] for e in ei]).astype(jnp.int32)
    vg = jnp.concatenate(
        [(a[:, 0] * N + a[:, 1]).astype(jnp.int32) for a in ea[:2]])
    vb = jnp.concatenate(
        [(a[:, 2] * N + a[:, 3]).astype(jnp.int32) for a in ea[:2]])
    gidx = jnp.concatenate([src, dst + N, vg, vb])

    w = weights
    tc1_w = []
    for r in RELS:
        for (W, b) in w['ef1_' + r]:
            tc1_w.append(W if W.shape[1] > 1 else W.reshape(1, H))
            tc1_w.append(b.reshape(1, H))
    tc1_w.append(w['nf1_input'][0][0].reshape(1, H))
    tc1_w.append(w['nf1_input'][0][1].reshape(1, H))

    ef1, table = _tc1(t_all, dc_voltages_input[:, None], dst[:, None], tc1_w)
    g = _scb(table, gidx)

    tc2_w = []
    for r in RELS:
        for (W, b) in w['ef2_' + r]:
            tc2_w.append(W)
            tc2_w.append(b.reshape(1, H))
    for (W, b) in w['ef2_self']:
        tc2_w.append(W)
        tc2_w.append(b.reshape(1, H))
    tc2_w.append(w['score'][0][0])
    tc2_w.append(w['score'][0][1].reshape(1, H))

    return _tc2(g, ef1, tc2_w)


# speedup vs baseline: 1.1979x; 1.0773x over previous
import functools
import jax
import jax.numpy as jnp
from jax import lax
from jax.experimental import pallas as pl
from jax.experimental.pallas import tpu as pltpu
from jax.experimental.pallas import tpu_sc as plsc

H = 128
RELS = ('nmos', 'pmos', 'R', 'L', 'C')
E_REL = (1024, 1024, 768, 640, 640)
OFF = (0, 1024, 2048, 2816, 3456)
E_TOT = 4096
N = 8192
NC, NS, L = 2, 16, 16
NW = NC * NS
SEG_OWN = N // NW
ACC_ROWS = SEG_OWN + 16
CHUNK = 128
N_CHUNKS = E_TOT // CHUNK
G_TOT = 3 * E_TOT
G_PER_TILE = G_TOT // (NC * NS)


def _gelu(x):
    return 0.5 * x * (1.0 + lax.erf(x * 0.7071067811865476))


SEG_BLK = 1024


def _tc1_body(t_ref, dc_ref, dstf_ref, *refs):
    w = refs[:32]
    ef1_ref, tab_ref = refs[32], refs[33]
    for r in range(5):
        w1, b1, w2, b2, w3, b3 = w[6 * r:6 * r + 6]
        x = t_ref[OFF[r]:OFF[r] + E_REL[r], :]
        h = _gelu(x * w1[...] + b1[...])
        h = _gelu(lax.dot_general(h, w2[...], (((1,), (1,)), ((), ())),
                                  preferred_element_type=jnp.float32) + b2[...])
        h = lax.dot_general(h, w3[...], (((1,), (1,)), ((), ())),
                            preferred_element_type=jnp.float32) + b3[...]
        ef1_ref[OFF[r]:OFF[r] + E_REL[r], :] = h
    ef1 = ef1_ref[...]
    g = lax.dot_general(ef1, ef1, (((0,), (0,)), ((), ())),
                        preferred_element_type=jnp.float32)
    att = lax.dot_general(ef1, g, (((1,), (0,)), ((), ())),
                          preferred_element_type=jnp.float32)
    ex = jnp.exp(att - jnp.max(att, axis=0, keepdims=True))
    p = jnp.concatenate([ex, ex * ef1], axis=1)
    win, bin_ = w[30], w[31]
    tab_ref[:N, :] = dc_ref[...] * win[...] + bin_[...]
    dsti = dstf_ref[...]
    base_iota = lax.broadcasted_iota(jnp.int32, (1, SEG_BLK), 1)
    pb = p.astype(jnp.bfloat16)

    def seg_block(b, _):
        oh = (dsti == base_iota + b * SEG_BLK).astype(jnp.bfloat16)
        seg = lax.dot_general(oh, pb, (((0,), (0,)), ((), ())),
                              preferred_element_type=jnp.float32)
        tab_ref[pl.ds(N + b * SEG_BLK, SEG_BLK), :] = (
            seg[:, H:] / (seg[:, :H] + 1e-16))
        return 0

    lax.fori_loop(0, N // SEG_BLK, seg_block, 0)


def _tc1(t_all, dc_in, dstf, wlist):
    outs = [
        jax.ShapeDtypeStruct((E_TOT, H), jnp.float32),
        jax.ShapeDtypeStruct((2 * N, H), jnp.float32),
    ]
    return pl.pallas_call(_tc1_body, out_shape=outs)(t_all, dc_in, dstf,
                                                     *wlist)


def _scb_body(tab_hbm, gidx_hbm, out_hbm, idxb, rows, sem):
    wid = lax.axis_index("s") * NC + lax.axis_index("c")
    base = wid * G_PER_TILE
    pltpu.sync_copy(gidx_hbm.at[pl.ds(base, G_PER_TILE)], idxb)
    pltpu.async_copy(tab_hbm.at[idxb], rows, sem).wait()
    pltpu.sync_copy(rows, out_hbm.at[pl.ds(base, G_PER_TILE), :])


def _scb(table, gidx):
    mesh = plsc.VectorSubcoreMesh(core_axis_name="c", subcore_axis_name="s",
                                  num_cores=NC, num_subcores=NS)
    f = pl.kernel(
        _scb_body,
        out_type=jax.ShapeDtypeStruct((G_TOT, H), jnp.float32),
        mesh=mesh,
        scratch_types=[
            pltpu.VMEM((G_PER_TILE,), jnp.int32),
            pltpu.VMEM((G_PER_TILE, H), jnp.float32),
            pltpu.SemaphoreType.DMA,
        ],
    )
    return f(table, gidx)


def _tc2_body(g_ref, ef1_ref, *refs):
    out_ref = refs[-1]
    w = refs[:-1]
    ef1 = ef1_ref[...]
    sw = w[40:48]
    h = _gelu(lax.dot_general(ef1, sw[0][...], (((1,), (1,)), ((), ())),
                              preferred_element_type=jnp.float32) + sw[1][...])
    h = _gelu(lax.dot_general(h, sw[2][...], (((1,), (1,)), ((), ())),
                              preferred_element_type=jnp.float32) + sw[3][...])
    h = _gelu(lax.dot_general(h, sw[4][...], (((1,), (1,)), ((), ())),
                              preferred_element_type=jnp.float32) + sw[5][...])
    selfo = lax.dot_general(h, sw[6][...], (((1,), (1,)), ((), ())),
                            preferred_element_type=jnp.float32) + sw[7][...]

    for r in range(5):
        w1, b1, w2, b2, w3, b3, w4, b4 = w[8 * r:8 * r + 8]
        lo, e = OFF[r], E_REL[r]
        src = g_ref[lo:lo + e, :]
        dst = g_ref[E_TOT + lo:E_TOT + lo + e, :]
        ef1_r = ef1_ref[lo:lo + e, :]
        w1m = w1[...]
        if r < 2:
            a = lax.dot_general(src, w1m[:, 0:H], (((1,), (1,)), ((), ())),
                                preferred_element_type=jnp.float32)
            a += lax.dot_general(dst, w1m[:, H:2 * H], (((1,), (1,)), ((), ())),
                                 preferred_element_type=jnp.float32)
            vg = g_ref[2 * E_TOT + lo:2 * E_TOT + lo + e, :]
            vb = g_ref[2 * E_TOT + 2048 + lo:2 * E_TOT + 2048 + lo + e, :]
            a += lax.dot_general(vg, w1m[:, 2 * H:3 * H], (((1,), (1,)), ((), ())),
                                 preferred_element_type=jnp.float32)
            a += lax.dot_general(vb, w1m[:, 3 * H:4 * H], (((1,), (1,)), ((), ())),
                                 preferred_element_type=jnp.float32)
            a += lax.dot_general(ef1_r, w1m[:, 4 * H:5 * H], (((1,), (1,)), ((), ())),
                                 preferred_element_type=jnp.float32)
        else:
            a = lax.dot_general(src, w1m[:, 0:H], (((1,), (1,)), ((), ())),
                                preferred_element_type=jnp.float32)
            a += lax.dot_general(dst, w1m[:, H:2 * H], (((1,), (1,)), ((), ())),
                                 preferred_element_type=jnp.float32)
            a += lax.dot_general(ef1_r, w1m[:, 2 * H:3 * H], (((1,), (1,)), ((), ())),
                                 preferred_element_type=jnp.float32)
        h = _gelu(a + b1[...])
        h = _gelu(lax.dot_general(h, w2[...], (((1,), (1,)), ((), ())),
                                  preferred_element_type=jnp.float32) + b2[...])
        h = _gelu(lax.dot_general(h, w3[...], (((1,), (1,)), ((), ())),
                                  preferred_element_type=jnp.float32) + b3[...])
        h = lax.dot_general(h, w4[...], (((1,), (1,)), ((), ())),
                            preferred_element_type=jnp.float32) + b4[...]
        ef2 = h + selfo[lo:lo + e, :]
        sw_, sb = w[48], w[49]
        out_ref[lo:lo + e, :] = lax.dot_general(
            ef2, sw_[...], (((1,), (1,)), ((), ())),
            preferred_element_type=jnp.float32) + sb[...]


def _tc2(g, ef1, wlist):
    return pl.pallas_call(
        _tc2_body,
        out_shape=jax.ShapeDtypeStruct((E_TOT, H), jnp.float32),
    )(g, ef1, *wlist)


@jax.jit
def kernel(edge_index_nmos, edge_index_pmos, edge_index_R, edge_index_L,
           edge_index_C, edge_attr_nmos, edge_attr_pmos, edge_attr_R,
           edge_attr_L, edge_attr_C, dc_voltages_input, dc_voltages_output,
           weights):
    ei = (edge_index_nmos, edge_index_pmos, edge_index_R, edge_index_L,
          edge_index_C)
    ea = (edge_attr_nmos, edge_attr_pmos, edge_attr_R, edge_attr_L,
          edge_attr_C)

    t_all = jnp.concatenate([a[:, -1:] for a in ea], axis=0)
    src = jnp.concatenate([e[0] for e in ei]).astype(jnp.int32)
    dst = jnp.concatenate([e[1] for e in ei]).astype(jnp.int32)
    vg = jnp.concatenate(
        [(a[:, 0] * N + a[:, 1]).astype(jnp.int32) for a in ea[:2]])
    vb = jnp.concatenate(
        [(a[:, 2] * N + a[:, 3]).astype(jnp.int32) for a in ea[:2]])
    gidx = jnp.concatenate([src, dst + N, vg, vb])

    w = weights
    tc1_w = []
    for r in RELS:
        for (W, b) in w['ef1_' + r]:
            tc1_w.append(W if W.shape[1] > 1 else W.reshape(1, H))
            tc1_w.append(b.reshape(1, H))
    tc1_w.append(w['nf1_input'][0][0].reshape(1, H))
    tc1_w.append(w['nf1_input'][0][1].reshape(1, H))

    ef1, table = _tc1(t_all, dc_voltages_input[:, None], dst[:, None], tc1_w)
    g = _scb(table, gidx)

    tc2_w = []
    for r in RELS:
        for (W, b) in w['ef2_' + r]:
            tc2_w.append(W)
            tc2_w.append(b.reshape(1, H))
    for (W, b) in w['ef2_self']:
        tc2_w.append(W)
        tc2_w.append(b.reshape(1, H))
    tc2_w.append(w['score'][0][0])
    tc2_w.append(w['score'][0][1].reshape(1, H))

    return _tc2(g, ef1, tc2_w)


# speedup vs baseline: 1.2248x; 1.0225x over previous
import functools
import jax
import jax.numpy as jnp
from jax import lax
from jax.experimental import pallas as pl
from jax.experimental.pallas import tpu as pltpu
from jax.experimental.pallas import tpu_sc as plsc

H = 128
RELS = ('nmos', 'pmos', 'R', 'L', 'C')
E_REL = (1024, 1024, 768, 640, 640)
OFF = (0, 1024, 2048, 2816, 3456)
E_TOT = 4096
N = 8192
NC, NS, L = 2, 16, 16
NW = NC * NS
SEG_OWN = N // NW
ACC_ROWS = SEG_OWN + 16
CHUNK = 128
N_CHUNKS = E_TOT // CHUNK
G_TOT = 3 * E_TOT
G_PER_TILE = G_TOT // (NC * NS)


def _gelu(x):
    return 0.5 * x * (1.0 + lax.erf(x * 0.7071067811865476))


SEG_BLK = 2048


def _tc1_body(t_ref, dc_ref, dstf_ref, *refs):
    w = refs[:32]
    ef1_ref, tab_ref = refs[32], refs[33]
    for r in range(5):
        w1, b1, w2, b2, w3, b3 = w[6 * r:6 * r + 6]
        x = t_ref[OFF[r]:OFF[r] + E_REL[r], :]
        h = _gelu(x * w1[...] + b1[...])
        h = _gelu(lax.dot_general(h, w2[...], (((1,), (1,)), ((), ())),
                                  preferred_element_type=jnp.float32) + b2[...])
        h = lax.dot_general(h, w3[...], (((1,), (1,)), ((), ())),
                            preferred_element_type=jnp.float32) + b3[...]
        ef1_ref[OFF[r]:OFF[r] + E_REL[r], :] = h
    ef1 = ef1_ref[...]
    g = lax.dot_general(ef1, ef1, (((0,), (0,)), ((), ())),
                        preferred_element_type=jnp.float32)
    att = lax.dot_general(ef1, g, (((1,), (0,)), ((), ())),
                          preferred_element_type=jnp.float32)
    ex = jnp.exp(att - jnp.max(att, axis=0, keepdims=True))
    p = jnp.concatenate([ex, ex * ef1], axis=1)
    win, bin_ = w[30], w[31]
    tab_ref[:N, :] = dc_ref[...] * win[...] + bin_[...]
    dsti = dstf_ref[...]
    base_iota = lax.broadcasted_iota(jnp.int32, (1, SEG_BLK), 1)
    pb = p.astype(jnp.bfloat16)

    def seg_block(b, _):
        oh = (dsti == base_iota + b * SEG_BLK).astype(jnp.bfloat16)
        seg = lax.dot_general(oh, pb, (((0,), (0,)), ((), ())),
                              preferred_element_type=jnp.float32)
        tab_ref[pl.ds(N + b * SEG_BLK, SEG_BLK), :] = (
            seg[:, H:] / (seg[:, :H] + 1e-16))
        return 0

    lax.fori_loop(0, N // SEG_BLK, seg_block, 0)


def _tc1(t_all, dc_in, dstf, wlist):
    outs = [
        jax.ShapeDtypeStruct((E_TOT, H), jnp.float32),
        jax.ShapeDtypeStruct((2 * N, H), jnp.float32),
    ]
    return pl.pallas_call(_tc1_body, out_shape=outs)(t_all, dc_in, dstf,
                                                     *wlist)


def _scb_body(tab_hbm, gidx_hbm, out_hbm, idxb, rows, sem):
    wid = lax.axis_index("s") * NC + lax.axis_index("c")
    base = wid * G_PER_TILE
    pltpu.sync_copy(gidx_hbm.at[pl.ds(base, G_PER_TILE)], idxb)
    pltpu.async_copy(tab_hbm.at[idxb], rows, sem).wait()
    pltpu.sync_copy(rows, out_hbm.at[pl.ds(base, G_PER_TILE), :])


def _scb(table, gidx):
    mesh = plsc.VectorSubcoreMesh(core_axis_name="c", subcore_axis_name="s",
                                  num_cores=NC, num_subcores=NS)
    f = pl.kernel(
        _scb_body,
        out_type=jax.ShapeDtypeStruct((G_TOT, H), jnp.float32),
        mesh=mesh,
        scratch_types=[
            pltpu.VMEM((G_PER_TILE,), jnp.int32),
            pltpu.VMEM((G_PER_TILE, H), jnp.float32),
            pltpu.SemaphoreType.DMA,
        ],
    )
    return f(table, gidx)


def _tc2_body(g_ref, ef1_ref, *refs):
    out_ref = refs[-1]
    w = refs[:-1]
    ef1 = ef1_ref[...]
    sw = w[40:48]
    h = _gelu(lax.dot_general(ef1, sw[0][...], (((1,), (1,)), ((), ())),
                              preferred_element_type=jnp.float32) + sw[1][...])
    h = _gelu(lax.dot_general(h, sw[2][...], (((1,), (1,)), ((), ())),
                              preferred_element_type=jnp.float32) + sw[3][...])
    h = _gelu(lax.dot_general(h, sw[4][...], (((1,), (1,)), ((), ())),
                              preferred_element_type=jnp.float32) + sw[5][...])
    selfo = lax.dot_general(h, sw[6][...], (((1,), (1,)), ((), ())),
                            preferred_element_type=jnp.float32) + sw[7][...]

    for r in range(5):
        w1, b1, w2, b2, w3, b3, w4, b4 = w[8 * r:8 * r + 8]
        lo, e = OFF[r], E_REL[r]
        src = g_ref[lo:lo + e, :]
        dst = g_ref[E_TOT + lo:E_TOT + lo + e, :]
        ef1_r = ef1_ref[lo:lo + e, :]
        w1m = w1[...]
        if r < 2:
            a = lax.dot_general(src, w1m[:, 0:H], (((1,), (1,)), ((), ())),
                                preferred_element_type=jnp.float32)
            a += lax.dot_general(dst, w1m[:, H:2 * H], (((1,), (1,)), ((), ())),
                                 preferred_element_type=jnp.float32)
            vg = g_ref[2 * E_TOT + lo:2 * E_TOT + lo + e, :]
            vb = g_ref[2 * E_TOT + 2048 + lo:2 * E_TOT + 2048 + lo + e, :]
            a += lax.dot_general(vg, w1m[:, 2 * H:3 * H], (((1,), (1,)), ((), ())),
                                 preferred_element_type=jnp.float32)
            a += lax.dot_general(vb, w1m[:, 3 * H:4 * H], (((1,), (1,)), ((), ())),
                                 preferred_element_type=jnp.float32)
            a += lax.dot_general(ef1_r, w1m[:, 4 * H:5 * H], (((1,), (1,)), ((), ())),
                                 preferred_element_type=jnp.float32)
        else:
            a = lax.dot_general(src, w1m[:, 0:H], (((1,), (1,)), ((), ())),
                                preferred_element_type=jnp.float32)
            a += lax.dot_general(dst, w1m[:, H:2 * H], (((1,), (1,)), ((), ())),
                                 preferred_element_type=jnp.float32)
            a += lax.dot_general(ef1_r, w1m[:, 2 * H:3 * H], (((1,), (1,)), ((), ())),
                                 preferred_element_type=jnp.float32)
        h = _gelu(a + b1[...])
        h = _gelu(lax.dot_general(h, w2[...], (((1,), (1,)), ((), ())),
                                  preferred_element_type=jnp.float32) + b2[...])
        h = _gelu(lax.dot_general(h, w3[...], (((1,), (1,)), ((), ())),
                                  preferred_element_type=jnp.float32) + b3[...])
        h = lax.dot_general(h, w4[...], (((1,), (1,)), ((), ())),
                            preferred_element_type=jnp.float32) + b4[...]
        ef2 = h + selfo[lo:lo + e, :]
        sw_, sb = w[48], w[49]
        out_ref[lo:lo + e, :] = lax.dot_general(
            ef2, sw_[...], (((1,), (1,)), ((), ())),
            preferred_element_type=jnp.float32) + sb[...]


def _tc2(g, ef1, wlist):
    return pl.pallas_call(
        _tc2_body,
        out_shape=jax.ShapeDtypeStruct((E_TOT, H), jnp.float32),
    )(g, ef1, *wlist)


@jax.jit
def kernel(edge_index_nmos, edge_index_pmos, edge_index_R, edge_index_L,
           edge_index_C, edge_attr_nmos, edge_attr_pmos, edge_attr_R,
           edge_attr_L, edge_attr_C, dc_voltages_input, dc_voltages_output,
           weights):
    ei = (edge_index_nmos, edge_index_pmos, edge_index_R, edge_index_L,
          edge_index_C)
    ea = (edge_attr_nmos, edge_attr_pmos, edge_attr_R, edge_attr_L,
          edge_attr_C)

    t_all = jnp.concatenate([a[:, -1:] for a in ea], axis=0)
    src = jnp.concatenate([e[0] for e in ei]).astype(jnp.int32)
    dst = jnp.concatenate([e[1] for e in ei]).astype(jnp.int32)
    vg = jnp.concatenate(
        [(a[:, 0] * N + a[:, 1]).astype(jnp.int32) for a in ea[:2]])
    vb = jnp.concatenate(
        [(a[:, 2] * N + a[:, 3]).astype(jnp.int32) for a in ea[:2]])
    gidx = jnp.concatenate([src, dst + N, vg, vb])

    w = weights
    tc1_w = []
    for r in RELS:
        for (W, b) in w['ef1_' + r]:
            tc1_w.append(W if W.shape[1] > 1 else W.reshape(1, H))
            tc1_w.append(b.reshape(1, H))
    tc1_w.append(w['nf1_input'][0][0].reshape(1, H))
    tc1_w.append(w['nf1_input'][0][1].reshape(1, H))

    ef1, table = _tc1(t_all, dc_voltages_input[:, None], dst[:, None], tc1_w)
    g = _scb(table, gidx)

    tc2_w = []
    for r in RELS:
        for (W, b) in w['ef2_' + r]:
            tc2_w.append(W)
            tc2_w.append(b.reshape(1, H))
    for (W, b) in w['ef2_self']:
        tc2_w.append(W)
        tc2_w.append(b.reshape(1, H))
    tc2_w.append(w['score'][0][0])
    tc2_w.append(w['score'][0][1].reshape(1, H))

    return _tc2(g, ef1, tc2_w)


# speedup vs baseline: 1.2327x; 1.0064x over previous
import functools
import jax
import jax.numpy as jnp
from jax import lax
from jax.experimental import pallas as pl
from jax.experimental.pallas import tpu as pltpu
from jax.experimental.pallas import tpu_sc as plsc

H = 128
RELS = ('nmos', 'pmos', 'R', 'L', 'C')
E_REL = (1024, 1024, 768, 640, 640)
OFF = (0, 1024, 2048, 2816, 3456)
E_TOT = 4096
N = 8192
NC, NS, L = 2, 16, 16
NW = NC * NS
SEG_OWN = N // NW
ACC_ROWS = SEG_OWN + 16
CHUNK = 128
N_CHUNKS = E_TOT // CHUNK
G_TOT = 3 * E_TOT
G_PER_TILE = G_TOT // (NC * NS)


def _gelu(x):
    return 0.5 * x * (1.0 + lax.erf(x * 0.7071067811865476))


SEG_BLK = 4096


def _tc1_body(t_ref, dc_ref, dstf_ref, *refs):
    w = refs[:32]
    ef1_ref, tab_ref = refs[32], refs[33]
    for r in range(5):
        w1, b1, w2, b2, w3, b3 = w[6 * r:6 * r + 6]
        x = t_ref[OFF[r]:OFF[r] + E_REL[r], :]
        h = _gelu(x * w1[...] + b1[...])
        h = _gelu(lax.dot_general(h, w2[...], (((1,), (1,)), ((), ())),
                                  preferred_element_type=jnp.float32) + b2[...])
        h = lax.dot_general(h, w3[...], (((1,), (1,)), ((), ())),
                            preferred_element_type=jnp.float32) + b3[...]
        ef1_ref[OFF[r]:OFF[r] + E_REL[r], :] = h
    ef1 = ef1_ref[...]
    g = lax.dot_general(ef1, ef1, (((0,), (0,)), ((), ())),
                        preferred_element_type=jnp.float32)
    att = lax.dot_general(ef1, g, (((1,), (0,)), ((), ())),
                          preferred_element_type=jnp.float32)
    ex = jnp.exp(att - jnp.max(att, axis=0, keepdims=True))
    p = jnp.concatenate([ex, ex * ef1], axis=1)
    win, bin_ = w[30], w[31]
    tab_ref[:N, :] = dc_ref[...] * win[...] + bin_[...]
    dsti = dstf_ref[...]
    base_iota = lax.broadcasted_iota(jnp.int32, (1, SEG_BLK), 1)
    pb = p.astype(jnp.bfloat16)

    def seg_block(b, _):
        oh = (dsti == base_iota + b * SEG_BLK).astype(jnp.bfloat16)
        seg = lax.dot_general(oh, pb, (((0,), (0,)), ((), ())),
                              preferred_element_type=jnp.float32)
        tab_ref[pl.ds(N + b * SEG_BLK, SEG_BLK), :] = (
            seg[:, H:] / (seg[:, :H] + 1e-16))
        return 0

    lax.fori_loop(0, N // SEG_BLK, seg_block, 0)


def _tc1(t_all, dc_in, dstf, wlist):
    outs = [
        jax.ShapeDtypeStruct((E_TOT, H), jnp.float32),
        jax.ShapeDtypeStruct((2 * N, H), jnp.float32),
    ]
    return pl.pallas_call(_tc1_body, out_shape=outs)(t_all, dc_in, dstf,
                                                     *wlist)


def _scb_body(tab_hbm, gidx_hbm, out_hbm, idxb, rows, sem):
    wid = lax.axis_index("s") * NC + lax.axis_index("c")
    base = wid * G_PER_TILE
    pltpu.sync_copy(gidx_hbm.at[pl.ds(base, G_PER_TILE)], idxb)
    pltpu.async_copy(tab_hbm.at[idxb], rows, sem).wait()
    pltpu.sync_copy(rows, out_hbm.at[pl.ds(base, G_PER_TILE), :])


def _scb(table, gidx):
    mesh = plsc.VectorSubcoreMesh(core_axis_name="c", subcore_axis_name="s",
                                  num_cores=NC, num_subcores=NS)
    f = pl.kernel(
        _scb_body,
        out_type=jax.ShapeDtypeStruct((G_TOT, H), jnp.float32),
        mesh=mesh,
        scratch_types=[
            pltpu.VMEM((G_PER_TILE,), jnp.int32),
            pltpu.VMEM((G_PER_TILE, H), jnp.float32),
            pltpu.SemaphoreType.DMA,
        ],
    )
    return f(table, gidx)


def _tc2_body(g_ref, ef1_ref, *refs):
    out_ref = refs[-1]
    w = refs[:-1]
    ef1 = ef1_ref[...]
    sw = w[40:48]
    h = _gelu(lax.dot_general(ef1, sw[0][...], (((1,), (1,)), ((), ())),
                              preferred_element_type=jnp.float32) + sw[1][...])
    h = _gelu(lax.dot_general(h, sw[2][...], (((1,), (1,)), ((), ())),
                              preferred_element_type=jnp.float32) + sw[3][...])
    h = _gelu(lax.dot_general(h, sw[4][...], (((1,), (1,)), ((), ())),
                              preferred_element_type=jnp.float32) + sw[5][...])
    selfo = lax.dot_general(h, sw[6][...], (((1,), (1,)), ((), ())),
                            preferred_element_type=jnp.float32) + sw[7][...]

    for r in range(5):
        w1, b1, w2, b2, w3, b3, w4, b4 = w[8 * r:8 * r + 8]
        lo, e = OFF[r], E_REL[r]
        src = g_ref[lo:lo + e, :]
        dst = g_ref[E_TOT + lo:E_TOT + lo + e, :]
        ef1_r = ef1_ref[lo:lo + e, :]
        w1m = w1[...]
        if r < 2:
            a = lax.dot_general(src, w1m[:, 0:H], (((1,), (1,)), ((), ())),
                                preferred_element_type=jnp.float32)
            a += lax.dot_general(dst, w1m[:, H:2 * H], (((1,), (1,)), ((), ())),
                                 preferred_element_type=jnp.float32)
            vg = g_ref[2 * E_TOT + lo:2 * E_TOT + lo + e, :]
            vb = g_ref[2 * E_TOT + 2048 + lo:2 * E_TOT + 2048 + lo + e, :]
            a += lax.dot_general(vg, w1m[:, 2 * H:3 * H], (((1,), (1,)), ((), ())),
                                 preferred_element_type=jnp.float32)
            a += lax.dot_general(vb, w1m[:, 3 * H:4 * H], (((1,), (1,)), ((), ())),
                                 preferred_element_type=jnp.float32)
            a += lax.dot_general(ef1_r, w1m[:, 4 * H:5 * H], (((1,), (1,)), ((), ())),
                                 preferred_element_type=jnp.float32)
        else:
            a = lax.dot_general(src, w1m[:, 0:H], (((1,), (1,)), ((), ())),
                                preferred_element_type=jnp.float32)
            a += lax.dot_general(dst, w1m[:, H:2 * H], (((1,), (1,)), ((), ())),
                                 preferred_element_type=jnp.float32)
            a += lax.dot_general(ef1_r, w1m[:, 2 * H:3 * H], (((1,), (1,)), ((), ())),
                                 preferred_element_type=jnp.float32)
        h = _gelu(a + b1[...])
        h = _gelu(lax.dot_general(h, w2[...], (((1,), (1,)), ((), ())),
                                  preferred_element_type=jnp.float32) + b2[...])
        h = _gelu(lax.dot_general(h, w3[...], (((1,), (1,)), ((), ())),
                                  preferred_element_type=jnp.float32) + b3[...])
        h = lax.dot_general(h, w4[...], (((1,), (1,)), ((), ())),
                            preferred_element_type=jnp.float32) + b4[...]
        ef2 = h + selfo[lo:lo + e, :]
        sw_, sb = w[48], w[49]
        out_ref[lo:lo + e, :] = lax.dot_general(
            ef2, sw_[...], (((1,), (1,)), ((), ())),
            preferred_element_type=jnp.float32) + sb[...]


def _tc2(g, ef1, wlist):
    return pl.pallas_call(
        _tc2_body,
        out_shape=jax.ShapeDtypeStruct((E_TOT, H), jnp.float32),
    )(g, ef1, *wlist)


@jax.jit
def kernel(edge_index_nmos, edge_index_pmos, edge_index_R, edge_index_L,
           edge_index_C, edge_attr_nmos, edge_attr_pmos, edge_attr_R,
           edge_attr_L, edge_attr_C, dc_voltages_input, dc_voltages_output,
           weights):
    ei = (edge_index_nmos, edge_index_pmos, edge_index_R, edge_index_L,
          edge_index_C)
    ea = (edge_attr_nmos, edge_attr_pmos, edge_attr_R, edge_attr_L,
          edge_attr_C)

    t_all = jnp.concatenate([a[:, -1:] for a in ea], axis=0)
    src = jnp.concatenate([e[0] for e in ei]).astype(jnp.int32)
    dst = jnp.concatenate([e[1] for e in ei]).astype(jnp.int32)
    vg = jnp.concatenate(
        [(a[:, 0] * N + a[:, 1]).astype(jnp.int32) for a in ea[:2]])
    vb = jnp.concatenate(
        [(a[:, 2] * N + a[:, 3]).astype(jnp.int32) for a in ea[:2]])
    gidx = jnp.concatenate([src, dst + N, vg, vb])

    w = weights
    tc1_w = []
    for r in RELS:
        for (W, b) in w['ef1_' + r]:
            tc1_w.append(W if W.shape[1] > 1 else W.reshape(1, H))
            tc1_w.append(b.reshape(1, H))
    tc1_w.append(w['nf1_input'][0][0].reshape(1, H))
    tc1_w.append(w['nf1_input'][0][1].reshape(1, H))

    ef1, table = _tc1(t_all, dc_voltages_input[:, None], dst[:, None], tc1_w)
    g = _scb(table, gidx)

    tc2_w = []
    for r in RELS:
        for (W, b) in w['ef2_' + r]:
            tc2_w.append(W)
            tc2_w.append(b.reshape(1, H))
    for (W, b) in w['ef2_self']:
        tc2_w.append(W)
        tc2_w.append(b.reshape(1, H))
    tc2_w.append(w['score'][0][0])
    tc2_w.append(w['score'][0][1].reshape(1, H))

    return _tc2(g, ef1, tc2_w)
